# baseline (speedup 1.0000x reference)
import functools

import jax
import jax.numpy as jnp
from jax import lax
from jax.experimental import pallas as pl
from jax.experimental.pallas import tpu as pltpu
from jax.experimental.pallas import tpu_sc as plsc

VOCAB = 10000
D = 300
DP = 304
NB = 4096
T = 50
NC, NS = 2, 16
NW = NC * NS
BCOL = NB // NW
HB = 64
NCHUNK = T * 2


def _sc_gather_t(table, data_t):
  mesh = plsc.VectorSubcoreMesh(core_axis_name="c", subcore_axis_name="s")

  @functools.partial(
      pl.kernel,
      mesh=mesh,
      out_type=jax.ShapeDtypeStruct((T, D, NB), jnp.float32),
      scratch_types=[
          pltpu.VMEM((T, BCOL), jnp.int32),
          pltpu.VMEM((2, HB, DP), jnp.float32),
          pltpu.VMEM((2, DP, HB), jnp.float32),
          pltpu.SemaphoreType.DMA,
          pltpu.SemaphoreType.DMA,
          pltpu.SemaphoreType.DMA,
          pltpu.SemaphoreType.DMA,
      ],
      compiler_params=pltpu.CompilerParams(
          use_tc_tiling_on_sc=False, needs_layout_passes=False
      ),
  )
  def k(table_hbm, idx_hbm, out_hbm, idx_v, rows_v, tr_v, g0, g1, s0, s1):
    gsem = (g0, g1)
    ssem = (s0, s1)
    wid = lax.axis_index("s") * NC + lax.axis_index("c")
    bbase = wid * BCOL
    pltpu.sync_copy(idx_hbm.at[:, pl.ds(bbase, BCOL)], idx_v)

    iota = jax.lax.iota(jnp.int32, 16)

    def gather(c, b):
      t = c // 2
      j = c % 2
      return pltpu.make_async_copy(
          table_hbm.at[idx_v.at[t, pl.ds(j * HB, HB)]], rows_v.at[b], gsem[b]
      )

    def write(c, b):
      t = c // 2
      j = c % 2
      return pltpu.make_async_copy(
          tr_v.at[b, pl.ds(0, D)],
          out_hbm.at[t, :, pl.ds(bbase + j * HB, HB)],
          ssem[b],
      )

    for b in range(2):
      gather(b, b).start()

    def body(p, carry):
      for b in range(2):
        c = 2 * p + b
        gather(c, b).wait()

        @pl.when(c >= 2)
        def _():
          write(c - 2, b).wait()

        src = rows_v.at[b]
        dst = tr_v.at[b]

        def row(r, cr):
          cc = jnp.full((16,), r, dtype=jnp.int32)
          for kv in range(DP // 16):
            v = src[r, pl.ds(16 * kv, 16)]
            plsc.store_scatter(dst, [iota + 16 * kv, cc], v)
          return cr

        lax.fori_loop(0, HB, row, 0)

        write(c, b).start()

        @pl.when(c + 2 < NCHUNK)
        def _():
          gather(c + 2, b).start()
      return carry

    lax.fori_loop(0, NCHUNK // 2, body, 0)
    for b in range(2):
      write(NCHUNK - 2 + b, b).wait()

  return k(table, data_t)


def kernel(data, ivectors):
  table = jnp.pad(ivectors, ((0, 0), (0, DP - D)))
  data_t = jnp.transpose(data.astype(jnp.int32))
  out_t = _sc_gather_t(table, data_t)
  return jnp.transpose(out_t, (2, 0, 1))


# speedup vs baseline: 2.3128x; 2.3128x over previous
import functools

import jax
import jax.numpy as jnp
from jax import lax
from jax.experimental import pallas as pl
from jax.experimental.pallas import tpu as pltpu
from jax.experimental.pallas import tpu_sc as plsc

VOCAB = 10000
D = 300
DP = 384
NB = 4096
T = 50
NC, NS = 2, 16
NW = NC * NS
SPW = NB // NW
HROWS = 80


def _sc_gather(table3, idx3):
  mesh = plsc.VectorSubcoreMesh(core_axis_name="c", subcore_axis_name="s")

  @functools.partial(
      pl.kernel,
      mesh=mesh,
      out_type=jax.ShapeDtypeStruct((NB, T, D), jnp.float32),
      scratch_types=[
          pltpu.VMEM((SPW, 2, HROWS), jnp.int32),
          pltpu.VMEM((2, 2 * HROWS, 128), jnp.float32),
          pltpu.VMEM((2, T, D), jnp.float32),
          pltpu.SemaphoreType.DMA,
          pltpu.SemaphoreType.DMA,
          pltpu.SemaphoreType.DMA,
          pltpu.SemaphoreType.DMA,
      ],
      compiler_params=pltpu.CompilerParams(
          use_tc_tiling_on_sc=True, needs_layout_passes=False
      ),
  )
  def k(tab_hbm, idx_hbm, out_hbm, idx_v, rows_v, slab_v, g0, g1, s0, s1):
    gsem = (g0, g1)
    ssem = (s0, s1)
    wid = lax.axis_index("s") * NC + lax.axis_index("c")
    sbase = wid * SPW
    pltpu.sync_copy(idx_hbm.at[pl.ds(sbase, SPW)], idx_v)

    iota = jax.lax.iota(jnp.int32, 16)
    cc_tail = 40 + (iota & 3)
    row4 = iota >> 2

    def gather(c, b, h):
      return pltpu.make_async_copy(
          tab_hbm.at[idx_v.at[c, h]],
          rows_v.at[b, pl.ds(h * HROWS, HROWS)],
          gsem[b],
      )

    def write(c, b):
      return pltpu.make_async_copy(
          slab_v.at[b], out_hbm.at[sbase + c], ssem[b]
      )

    for b in range(2):
      for h in range(2):
        gather(b, b, h).start()

    def body(p, carry):
      for b in range(2):
        c = 2 * p + b
        gather(c, b, 0).wait()
        gather(c, b, 1).wait()

        @pl.when(c >= 2)
        def _():
          write(c - 2, b).wait()

        src = rows_v.at[b]
        dst = slab_v.at[b]
        for t in range(T):
          r0 = HROWS * (t // 25) + 3 * (t % 25)
          for kv in range(18):
            dst[t, pl.ds(16 * kv, 16)] = src[r0 + kv // 8, pl.ds((16 * kv) % 128, 16)]
          dst[t, pl.ds(280, 16)] = src[r0 + 2, pl.ds(24, 16)]
        for g in range(13):
          tt = row4 + 4 * g
          rr = HROWS * (tt // 25) + 3 * (tt % 25) + 2
          mask = iota < 8 if g == 12 else None
          vals = plsc.load_gather(src, [rr, cc_tail], mask=mask)
          plsc.store_scatter(dst, [tt, 296 + (iota & 3)], vals, mask=mask)

        write(c, b).start()

        @pl.when(c + 2 < SPW)
        def _():
          gather(c + 2, b, 0).start()
          gather(c + 2, b, 1).start()
      return carry

    lax.fori_loop(0, SPW // 2, body, 0)
    for b in range(2):
      write(SPW - 2 + b, b).wait()

  return k(table3, idx3)


def kernel(data, ivectors):
  table3 = jnp.pad(ivectors, ((0, 0), (0, DP - D))).reshape(VOCAB * 3, 128)
  d2 = data.astype(jnp.int32).reshape(NB, 2, 25)
  mm = jnp.minimum(jnp.arange(HROWS) // 3, 24)
  jj = jnp.arange(HROWS) % 3
  idx3 = 3 * jnp.take(d2, mm, axis=2) + jj[None, None, :]
  return _sc_gather(table3, idx3)
